# Initial kernel scaffold; baseline (speedup 1.0000x reference)
import jax
import jax.numpy as jnp
from jax import lax
from jax.experimental import pallas as pl
from jax.experimental.pallas import tpu as pltpu

GAMMA = 0.01618
K_STATIC = 128
NEG_SLOPE = 0.01
OUT_DIM = 8192
ROWS = 8
COLS = OUT_DIM // ROWS
SIGN32 = jnp.int32(-2147483648)


def _mm_kernel(x_ref, w_ref, b_ref, o_ref):
    h = lax.dot_general(
        x_ref[...], w_ref[...],
        dimension_numbers=(((1,), (1,)), ((), ())),
        preferred_element_type=jnp.float32,
    )
    h = h + b_ref[...]
    o_ref[...] = jnp.where(h >= 0, h, NEG_SLOPE * h)


def _matmul(x, W, b2):
    tj = 1024
    grid = OUT_DIM // tj
    return pl.pallas_call(
        _mm_kernel,
        grid=(grid,),
        in_specs=[
            pl.BlockSpec((64, 2048), lambda j: (0, 0)),
            pl.BlockSpec((tj, 2048), lambda j: (j, 0)),
            pl.BlockSpec((1, tj), lambda j: (0, j)),
        ],
        out_specs=pl.BlockSpec((64, tj), lambda j: (0, j)),
        out_shape=jax.ShapeDtypeStruct((64, OUT_DIM), jnp.float32),
    )(x, W, b2)


def _scan_kernel(h_ref, o_ref, phi_ref):
    t = pl.program_id(0)

    @pl.when(t == 0)
    def _():
        phi_ref[...] = jnp.ones_like(phi_ref)

    phi = phi_ref[...]
    s = h_ref[0] * phi

    ibits = lax.bitcast_convert_type(s, jnp.int32)
    skey = ibits ^ (lax.shift_right_arithmetic(ibits, 31) & jnp.int32(0x7FFFFFFF))

    def vbody(i, tu):
        cand = tu | lax.shift_left(jnp.int32(1), 31 - i)
        thresh = cand ^ SIGN32
        cnt = jnp.sum((skey >= thresh).astype(jnp.int32))
        return lax.select(cnt >= K_STATIC, cand, tu)

    tu = lax.fori_loop(0, 32, vbody, jnp.int32(0))
    T = tu ^ SIGN32

    gt = skey > T
    eq = skey == T
    need = K_STATIC - jnp.sum(gt.astype(jnp.int32))

    idx = (lax.broadcasted_iota(jnp.int32, (ROWS, COLS), 0) * COLS
           + lax.broadcasted_iota(jnp.int32, (ROWS, COLS), 1))

    def ibody(i, p):
        cand = p + lax.shift_left(jnp.int32(1), 12 - i)
        cnt = jnp.sum((eq & (idx < cand)).astype(jnp.int32))
        return lax.select(cnt >= need, p, cand)

    m = lax.fori_loop(0, 13, ibody, jnp.int32(0))

    maskf = (gt | (eq & (idx <= m))).astype(jnp.float32)
    o_ref[0] = maskf

    phi = jnp.where(phi < 1.0, phi + GAMMA, phi)
    phi = jnp.where(phi >= 1.0, 1.0, phi)
    phi_ref[...] = phi * (1.0 - maskf)


def kernel(x, W, b, k):
    h = _matmul(x, W, b.reshape(1, OUT_DIM))
    hr = h.reshape(64, ROWS, COLS)
    mask = pl.pallas_call(
        _scan_kernel,
        grid=(64,),
        in_specs=[pl.BlockSpec((1, ROWS, COLS), lambda t: (t, 0, 0))],
        out_specs=pl.BlockSpec((1, ROWS, COLS), lambda t: (t, 0, 0)),
        out_shape=jax.ShapeDtypeStruct((64, ROWS, COLS), jnp.float32),
        scratch_shapes=[pltpu.VMEM((ROWS, COLS), jnp.float32)],
    )(hr)
    k_unit = jnp.asarray(k // k, jnp.float32)
    return mask.reshape(64, OUT_DIM) * k_unit


# speedup vs baseline: 1.5236x; 1.5236x over previous
import jax
import jax.numpy as jnp
from jax import lax
from jax.experimental import pallas as pl
from jax.experimental.pallas import tpu as pltpu

GAMMA = 0.01618
K_STATIC = 128
NEG_SLOPE = 0.01
OUT_DIM = 8192
ROWS = 8
COLS = OUT_DIM // ROWS
SIGN32 = -2147483648


def _mm_kernel(x_ref, w_ref, b_ref, o_ref):
    h = lax.dot_general(
        x_ref[...], w_ref[...],
        dimension_numbers=(((1,), (1,)), ((), ())),
        preferred_element_type=jnp.float32,
    )
    h = h + b_ref[...]
    o_ref[...] = jnp.where(h >= 0, h, NEG_SLOPE * h)


def _matmul(x, W, b2):
    tj = 1024
    grid = OUT_DIM // tj
    return pl.pallas_call(
        _mm_kernel,
        grid=(grid,),
        in_specs=[
            pl.BlockSpec((64, 2048), lambda j: (0, 0)),
            pl.BlockSpec((tj, 2048), lambda j: (j, 0)),
            pl.BlockSpec((1, tj), lambda j: (0, j)),
        ],
        out_specs=pl.BlockSpec((64, tj), lambda j: (0, j)),
        out_shape=jax.ShapeDtypeStruct((64, OUT_DIM), jnp.float32),
    )(x, W, b2)


def _scan_kernel(h_ref, o_ref, phi_ref):
    t = pl.program_id(0)

    @pl.when(t == 0)
    def _():
        phi_ref[...] = jnp.ones_like(phi_ref)

    sign32 = jnp.int32(SIGN32)
    phi = phi_ref[...]
    s = h_ref[0] * phi

    ibits = lax.bitcast_convert_type(s, jnp.int32)
    skey = ibits ^ (lax.shift_right_arithmetic(ibits, 31) & jnp.int32(0x7FFFFFFF))

    def vbody(i, tu):
        cand = tu | lax.shift_left(jnp.int32(1), 31 - i)
        thresh = cand ^ sign32
        cnt = jnp.sum((skey >= thresh).astype(jnp.int32))
        return lax.select(cnt >= K_STATIC, cand, tu)

    tu = lax.fori_loop(0, 32, vbody, jnp.int32(0))
    T = tu ^ sign32

    gt = skey > T
    eq = skey == T
    need = K_STATIC - jnp.sum(gt.astype(jnp.int32))

    idx = (lax.broadcasted_iota(jnp.int32, (ROWS, COLS), 0) * COLS
           + lax.broadcasted_iota(jnp.int32, (ROWS, COLS), 1))

    def ibody(i, p):
        cand = p + lax.shift_left(jnp.int32(1), 12 - i)
        cnt = jnp.sum((eq & (idx < cand)).astype(jnp.int32))
        return lax.select(cnt >= need, p, cand)

    m = lax.fori_loop(0, 13, ibody, jnp.int32(0))

    maskf = (gt | (eq & (idx <= m))).astype(jnp.float32)
    o_ref[0] = maskf

    phi = jnp.where(phi < 1.0, phi + GAMMA, phi)
    phi = jnp.where(phi >= 1.0, 1.0, phi)
    phi_ref[...] = phi * (1.0 - maskf)


def kernel(x, W, b, k):
    h = _matmul(x, W, b.reshape(1, OUT_DIM))
    hr = h.reshape(64, ROWS, COLS)
    mask = pl.pallas_call(
        _scan_kernel,
        grid=(64,),
        in_specs=[pl.BlockSpec((1, ROWS, COLS), lambda t: (t, 0, 0))],
        out_specs=pl.BlockSpec((1, ROWS, COLS), lambda t: (t, 0, 0)),
        out_shape=jax.ShapeDtypeStruct((64, ROWS, COLS), jnp.float32),
        scratch_shapes=[pltpu.VMEM((ROWS, COLS), jnp.float32)],
    )(hr)
    k_unit = jnp.asarray(k // k, jnp.float32)
    return mask.reshape(64, OUT_DIM) * k_unit


# speedup vs baseline: 1.5855x; 1.0406x over previous
import jax
import jax.numpy as jnp
from jax import lax
from jax.experimental import pallas as pl
from jax.experimental.pallas import tpu as pltpu

GAMMA = 0.01618
K_STATIC = 128
NEG_SLOPE = 0.01
OUT_DIM = 8192
ROWS = 8
COLS = OUT_DIM // ROWS
SIGN32 = -2147483648


def _mm_kernel(x_ref, w_ref, b_ref, o_ref):
    h = lax.dot_general(
        x_ref[...], w_ref[...],
        dimension_numbers=(((1,), (1,)), ((), ())),
        preferred_element_type=jnp.float32,
    )
    h = h + b_ref[...]
    o_ref[...] = jnp.where(h >= 0, h, NEG_SLOPE * h)


def _matmul(x, W, b2):
    tj = 1024
    grid = OUT_DIM // tj
    return pl.pallas_call(
        _mm_kernel,
        grid=(grid,),
        in_specs=[
            pl.BlockSpec((64, 2048), lambda j: (0, 0)),
            pl.BlockSpec((tj, 2048), lambda j: (j, 0)),
            pl.BlockSpec((1, tj), lambda j: (0, j)),
        ],
        out_specs=pl.BlockSpec((64, tj), lambda j: (0, j)),
        out_shape=jax.ShapeDtypeStruct((64, OUT_DIM), jnp.float32),
    )(x, W, b2)


def _scan_kernel(h_ref, o_ref, phi_ref):
    t = pl.program_id(0)

    @pl.when(t == 0)
    def _():
        phi_ref[...] = jnp.ones_like(phi_ref)

    sign32 = jnp.full((1, 1), SIGN32, jnp.int32)
    phi = phi_ref[...]
    s = h_ref[0] * phi

    ibits = lax.bitcast_convert_type(s, jnp.int32)
    skey = ibits ^ (lax.shift_right_arithmetic(ibits, 31) & jnp.int32(0x7FFFFFFF))

    tu = jnp.zeros((1, 1), jnp.int32)
    for b in range(31, -1, -1):
        cand = tu | jnp.int32((1 << b) if b < 31 else SIGN32)
        pred = (skey >= (cand ^ sign32)).astype(jnp.int32)
        cnt = jnp.sum(pred, axis=(0, 1), keepdims=True)
        tu = jnp.where(cnt >= K_STATIC, cand, tu)
    T = tu ^ sign32

    gt = skey > T
    eq = skey == T
    need = K_STATIC - jnp.sum(gt.astype(jnp.int32), axis=(0, 1), keepdims=True)

    idx = (lax.broadcasted_iota(jnp.int32, (ROWS, COLS), 0) * COLS
           + lax.broadcasted_iota(jnp.int32, (ROWS, COLS), 1))
    eqi = eq.astype(jnp.int32)

    m = jnp.zeros((1, 1), jnp.int32)
    for b in range(12, -1, -1):
        cand = m + jnp.int32(1 << b)
        pred = eqi & (idx < cand).astype(jnp.int32)
        cnt = jnp.sum(pred, axis=(0, 1), keepdims=True)
        m = jnp.where(cnt >= need, m, cand)

    maskf = (gt | (eq & (idx <= m))).astype(jnp.float32)
    o_ref[0] = maskf

    phi = jnp.where(phi < 1.0, phi + GAMMA, phi)
    phi = jnp.where(phi >= 1.0, 1.0, phi)
    phi_ref[...] = phi * (1.0 - maskf)


def kernel(x, W, b, k):
    h = _matmul(x, W, b.reshape(1, OUT_DIM))
    hr = h.reshape(64, ROWS, COLS)
    mask = pl.pallas_call(
        _scan_kernel,
        grid=(64,),
        in_specs=[pl.BlockSpec((1, ROWS, COLS), lambda t: (t, 0, 0))],
        out_specs=pl.BlockSpec((1, ROWS, COLS), lambda t: (t, 0, 0)),
        out_shape=jax.ShapeDtypeStruct((64, ROWS, COLS), jnp.float32),
        scratch_shapes=[pltpu.VMEM((ROWS, COLS), jnp.float32)],
    )(hr)
    k_unit = jnp.asarray(k // k, jnp.float32)
    return mask.reshape(64, OUT_DIM) * k_unit


# speedup vs baseline: 3.5469x; 2.2370x over previous
import jax
import jax.numpy as jnp
from jax import lax
from jax.experimental import pallas as pl
from jax.experimental.pallas import tpu as pltpu

GAMMA = 0.01618
K_STATIC = 128
NEG_SLOPE = 0.01
OUT_DIM = 8192
ROWS = 8
COLS = OUT_DIM // ROWS
SIGN32 = -2147483648


def _mm_kernel(x_ref, w_ref, b_ref, o_ref):
    h = lax.dot_general(
        x_ref[...], w_ref[...],
        dimension_numbers=(((1,), (1,)), ((), ())),
        preferred_element_type=jnp.float32,
    )
    h = h + b_ref[...]
    o_ref[...] = jnp.where(h >= 0, h, NEG_SLOPE * h)


def _matmul(x, W, b2):
    tj = 1024
    grid = OUT_DIM // tj
    return pl.pallas_call(
        _mm_kernel,
        grid=(grid,),
        in_specs=[
            pl.BlockSpec((64, 2048), lambda j: (0, 0)),
            pl.BlockSpec((tj, 2048), lambda j: (j, 0)),
            pl.BlockSpec((1, tj), lambda j: (0, j)),
        ],
        out_specs=pl.BlockSpec((64, tj), lambda j: (0, j)),
        out_shape=jax.ShapeDtypeStruct((64, OUT_DIM), jnp.float32),
    )(x, W, b2)


def _scan_kernel(h_ref, o_ref, phi_ref):
    t = pl.program_id(0)

    @pl.when(t == 0)
    def _():
        phi_ref[...] = jnp.ones_like(phi_ref)

    sign32 = jnp.full((1, 1), SIGN32, jnp.int32)
    phi = phi_ref[...]
    s = h_ref[0] * phi

    ibits = lax.bitcast_convert_type(s, jnp.int32)
    skey = ibits ^ (lax.shift_right_arithmetic(ibits, 31) & jnp.int32(0x7FFFFFFF))

    tu = jnp.zeros((1, 1), jnp.int32)
    for shift in range(28, -1, -4):
        digit = jnp.zeros((1, 1), jnp.int32)
        for j in range(1, 16):
            cand = tu | lax.shift_left(jnp.full((1, 1), j, jnp.int32),
                                       shift)
            pred = (skey >= (cand ^ sign32)).astype(jnp.int32)
            cnt = jnp.sum(pred, axis=(0, 1), keepdims=True)
            digit += (cnt >= K_STATIC).astype(jnp.int32)
        tu = tu | lax.shift_left(digit, shift)
    T = tu ^ sign32

    gt = skey > T
    eq = skey == T
    need = K_STATIC - jnp.sum(gt.astype(jnp.int32), axis=(0, 1), keepdims=True)

    idx = (lax.broadcasted_iota(jnp.int32, (ROWS, COLS), 0) * COLS
           + lax.broadcasted_iota(jnp.int32, (ROWS, COLS), 1))
    eqi = eq.astype(jnp.int32)

    m = jnp.zeros((1, 1), jnp.int32)
    for shift in (9, 5, 1, 0):
        top = 16 if shift else 2
        digit = jnp.zeros((1, 1), jnp.int32)
        for j in range(1, top):
            cand = m + jnp.int32(j << shift)
            pred = eqi & (idx < cand).astype(jnp.int32)
            cnt = jnp.sum(pred, axis=(0, 1), keepdims=True)
            digit += (cnt < need).astype(jnp.int32)
        m = m + lax.shift_left(digit, shift)

    maskf = (gt | (eq & (idx <= m))).astype(jnp.float32)
    o_ref[0] = maskf

    phi = jnp.where(phi < 1.0, phi + GAMMA, phi)
    phi = jnp.where(phi >= 1.0, 1.0, phi)
    phi_ref[...] = phi * (1.0 - maskf)


def kernel(x, W, b, k):
    h = _matmul(x, W, b.reshape(1, OUT_DIM))
    hr = h.reshape(64, ROWS, COLS)
    mask = pl.pallas_call(
        _scan_kernel,
        grid=(64,),
        in_specs=[pl.BlockSpec((1, ROWS, COLS), lambda t: (t, 0, 0))],
        out_specs=pl.BlockSpec((1, ROWS, COLS), lambda t: (t, 0, 0)),
        out_shape=jax.ShapeDtypeStruct((64, ROWS, COLS), jnp.float32),
        scratch_shapes=[pltpu.VMEM((ROWS, COLS), jnp.float32)],
    )(hr)
    k_unit = jnp.asarray(k // k, jnp.float32)
    return mask.reshape(64, OUT_DIM) * k_unit


# speedup vs baseline: 4.4374x; 1.2511x over previous
import jax
import jax.numpy as jnp
from jax import lax
from jax.experimental import pallas as pl
from jax.experimental.pallas import tpu as pltpu

GAMMA = 0.01618
K_STATIC = 128
NEG_SLOPE = 0.01
OUT_DIM = 8192
ROWS = 8
COLS = OUT_DIM // ROWS
SIGN32 = -2147483648


def _mm_kernel(x_ref, w_ref, b_ref, o_ref):
    h = lax.dot_general(
        x_ref[...], w_ref[...],
        dimension_numbers=(((1,), (1,)), ((), ())),
        preferred_element_type=jnp.float32,
    )
    h = h + b_ref[...]
    o_ref[...] = jnp.where(h >= 0, h, NEG_SLOPE * h)


def _matmul(x, W, b2):
    tj = 1024
    grid = OUT_DIM // tj
    return pl.pallas_call(
        _mm_kernel,
        grid=(grid,),
        in_specs=[
            pl.BlockSpec((64, 2048), lambda j: (0, 0)),
            pl.BlockSpec((tj, 2048), lambda j: (j, 0)),
            pl.BlockSpec((1, tj), lambda j: (0, j)),
        ],
        out_specs=pl.BlockSpec((64, tj), lambda j: (0, j)),
        out_shape=jax.ShapeDtypeStruct((64, OUT_DIM), jnp.float32),
    )(x, W, b2)


def _scan_kernel(h_ref, o_ref):
    sign32 = jnp.full((1, 1), SIGN32, jnp.int32)
    idx = (lax.broadcasted_iota(jnp.int32, (ROWS, COLS), 0) * COLS
           + lax.broadcasted_iota(jnp.int32, (ROWS, COLS), 1))

    def step(t, phi):
        s = h_ref[t] * phi

        ibits = lax.bitcast_convert_type(s, jnp.int32)
        skey = ibits ^ (lax.shift_right_arithmetic(ibits, 31)
                        & jnp.int32(0x7FFFFFFF))

        tu = jnp.zeros((1, 1), jnp.int32)
        for shift in range(28, -1, -4):
            digit = jnp.zeros((1, 1), jnp.int32)
            for j in range(1, 16):
                cand = tu | lax.shift_left(jnp.full((1, 1), j, jnp.int32),
                                           shift)
                pred = (skey >= (cand ^ sign32)).astype(jnp.int32)
                cnt = jnp.sum(pred, axis=(0, 1), keepdims=True)
                digit += (cnt >= K_STATIC).astype(jnp.int32)
            tu = tu | lax.shift_left(digit, shift)
        T = tu ^ sign32

        gt = skey > T
        eq = skey == T
        n_gt = jnp.sum(gt.astype(jnp.int32), axis=(0, 1), keepdims=True)
        need = K_STATIC - n_gt
        n_eq = jnp.sum(eq.astype(jnp.int32), axis=(0, 1), keepdims=True)

        def no_tie():
            return (skey >= T).astype(jnp.float32)

        def tie_break():
            eqi = eq.astype(jnp.int32)
            m = jnp.zeros((1, 1), jnp.int32)
            for shift in (9, 5, 1, 0):
                top = 16 if shift else 2
                digit = jnp.zeros((1, 1), jnp.int32)
                for j in range(1, top):
                    cand = m + jnp.int32(j << shift)
                    pred = eqi & (idx < cand).astype(jnp.int32)
                    cnt = jnp.sum(pred, axis=(0, 1), keepdims=True)
                    digit += (cnt < need).astype(jnp.int32)
                m = m + lax.shift_left(digit, shift)
            return (gt | (eq & (idx <= m))).astype(jnp.float32)

        maskf = lax.cond(n_eq[0, 0] == need[0, 0], no_tie, tie_break)
        o_ref[t] = maskf

        phi = jnp.where(phi < 1.0, phi + GAMMA, phi)
        phi = jnp.where(phi >= 1.0, 1.0, phi)
        return phi * (1.0 - maskf)

    lax.fori_loop(0, 64, step, jnp.ones((ROWS, COLS), jnp.float32))


def kernel(x, W, b, k):
    h = _matmul(x, W, b.reshape(1, OUT_DIM))
    hr = h.reshape(64, ROWS, COLS)
    mask = pl.pallas_call(
        _scan_kernel,
        in_specs=[pl.BlockSpec((64, ROWS, COLS), lambda: (0, 0, 0))],
        out_specs=pl.BlockSpec((64, ROWS, COLS), lambda: (0, 0, 0)),
        out_shape=jax.ShapeDtypeStruct((64, ROWS, COLS), jnp.float32),
    )(hr)
    k_unit = jnp.asarray(k // k, jnp.float32)
    return mask.reshape(64, OUT_DIM) * k_unit
